# Initial kernel scaffold; baseline (speedup 1.0000x reference)
import jax
import jax.numpy as jnp
from jax.experimental import pallas as pl


def kernel(categorical_inputs, numerical_inputs, emb_tables, W1, b1, g1, be1, W2, b2, g2, be2, W3, b3, g3, be3, W4, b4):
    raise NotImplementedError("write your pallas kernel here")


# speedup vs baseline: 3.9482x; 3.9482x over previous
import functools
import math

import jax
import jax.numpy as jnp
from jax import lax
from jax.experimental import pallas as pl
from jax.experimental.pallas import tpu as pltpu
from jax.experimental.pallas import tpu_sc as plsc

_B = 16384
_NF = 26
_VOCAB = 100000
_ED = 50
_EPS = 1e-5
_INV = 1.0 / math.sqrt(1.0 + _EPS)

_ROWS = _B * _NF
_NC, _NS = 2, 16
_NW = _NC * _NS
_RPW = _ROWS // _NW
_CHUNK = 128
_NCH = _RPW // _CHUNK
_K = 8

def _sc_gather_body(table_hbm, idx_hbm, out_hbm, idx_v, rows_v, gsem, ssem):
    wid = lax.axis_index("s") * _NC + lax.axis_index("c")
    pltpu.sync_copy(idx_hbm.at[pl.ds(wid * _NCH, _NCH)], idx_v)

    def group(g, carry):
        c0 = g * _K
        gd = []
        for b in range(_K):
            gd.append(pltpu.async_copy(
                table_hbm.at[idx_v.at[c0 + b]], rows_v.at[b], gsem))
        sd = []
        for b in range(_K):
            gd[b].wait()
            sd.append(pltpu.async_copy(
                rows_v.at[b],
                out_hbm.at[pl.ds((wid * _NCH + c0 + b) * _CHUNK, _CHUNK)],
                ssem))
        for b in range(_K):
            sd[b].wait()
        return carry

    lax.fori_loop(0, _NCH // _K, group, 0)


@functools.cache
def _sc_gather():
    mesh = plsc.VectorSubcoreMesh(
        core_axis_name="c", subcore_axis_name="s",
        num_cores=_NC, num_subcores=_NS)
    return pl.kernel(
        _sc_gather_body,
        out_type=jax.ShapeDtypeStruct((_ROWS, _ED), jnp.float32),
        mesh=mesh,
        scratch_types=[
            pltpu.VMEM((_NCH, _CHUNK), jnp.int32),
            pltpu.VMEM((_K, _CHUNK, _ED), jnp.float32),
            pltpu.SemaphoreType.DMA,
            pltpu.SemaphoreType.DMA,
        ],
        compiler_params=pltpu.CompilerParams(use_tc_tiling_on_sc=False),
    )


_BLK = 512
_E1 = _NF * _ED
_ND = 13
_H1, _H2, _H3 = 512, 256, 128


def _mlp_body(emb_ref, num_ref, w1e_ref, w1n_ref, b1_ref, g1_ref, be1_ref,
              w2_ref, b2_ref, g2_ref, be2_ref,
              w3_ref, b3_ref, g3_ref, be3_ref,
              w4_ref, b4_ref, out_ref):
    h = jnp.dot(emb_ref[...], w1e_ref[...], preferred_element_type=jnp.float32)
    h = h + jnp.dot(num_ref[...], w1n_ref[...],
                    preferred_element_type=jnp.float32)
    h = ((h + b1_ref[...]) * _INV) * g1_ref[...] + be1_ref[...]
    x = jnp.maximum(h, 0.0)

    h = jnp.dot(x, w2_ref[...], preferred_element_type=jnp.float32)
    h = ((h + b2_ref[...]) * _INV) * g2_ref[...] + be2_ref[...]
    x = jnp.maximum(h, 0.0)

    h = jnp.dot(x, w3_ref[...], preferred_element_type=jnp.float32)
    h = ((h + b3_ref[...]) * _INV) * g3_ref[...] + be3_ref[...]
    x = jnp.maximum(h, 0.0)

    z = jnp.dot(x, w4_ref[...], preferred_element_type=jnp.float32)
    z = z + b4_ref[...]
    out_ref[...] = jax.nn.sigmoid(z)


def _full(shape):
    return pl.BlockSpec(shape, lambda i: (0, 0))


_mlp_call = pl.pallas_call(
    _mlp_body,
    grid=(_B // _BLK,),
    in_specs=[
        pl.BlockSpec((_BLK, _E1), lambda i: (i, 0)),
        pl.BlockSpec((_BLK, _ND), lambda i: (i, 0)),
        _full((_E1, _H1)), _full((_ND, _H1)),
        _full((1, _H1)), _full((1, _H1)), _full((1, _H1)),
        _full((_H1, _H2)), _full((1, _H2)), _full((1, _H2)), _full((1, _H2)),
        _full((_H2, _H3)), _full((1, _H3)), _full((1, _H3)), _full((1, _H3)),
        _full((_H3, 1)), _full((1, 1)),
    ],
    out_specs=pl.BlockSpec((_BLK, 1), lambda i: (i, 0)),
    out_shape=jax.ShapeDtypeStruct((_B, 1), jnp.float32),
)


def kernel(categorical_inputs, numerical_inputs, emb_tables,
           W1, b1, g1, be1, W2, b2, g2, be2, W3, b3, g3, be3, W4, b4):
    cat = jnp.clip(categorical_inputs, 0, _VOCAB - 1).astype(jnp.int32)
    idx = cat + (jnp.arange(_NF, dtype=jnp.int32) * _VOCAB)[None, :]
    idx2d = idx.reshape(_NW * _NCH, _CHUNK)
    table_flat = emb_tables.reshape(_NF * _VOCAB, _ED)

    emb_flat = _sc_gather()(table_flat, idx2d)
    emb = emb_flat.reshape(_B, _E1)

    w1 = W1.T
    out = _mlp_call(
        emb, numerical_inputs,
        w1[:_E1], w1[_E1:],
        b1.reshape(1, _H1), g1.reshape(1, _H1), be1.reshape(1, _H1),
        W2.T, b2.reshape(1, _H2), g2.reshape(1, _H2), be2.reshape(1, _H2),
        W3.T, b3.reshape(1, _H3), g3.reshape(1, _H3), be3.reshape(1, _H3),
        W4.T, b4.reshape(1, 1),
    )
    return out[:, 0]


# speedup vs baseline: 6.4068x; 1.6227x over previous
import functools
import math

import jax
import jax.numpy as jnp
from jax import lax
from jax.experimental import pallas as pl
from jax.experimental.pallas import tpu as pltpu
from jax.experimental.pallas import tpu_sc as plsc

_B = 16384
_NF = 26
_VOCAB = 100000
_ED = 50
_EDP = 128
_EPS = 1e-5
_INV = 1.0 / math.sqrt(1.0 + _EPS)

_ROWS = _B * _NF
_NC, _NS = 2, 16
_NW = _NC * _NS
_RPW = _ROWS // _NW
_CHUNK = 128
_NCH = _RPW // _CHUNK
_K = 4


def _sc_gather_body(table_hbm, idx_hbm, out_hbm, idx_v, rows_v, gsem, ssem):
    wid = lax.axis_index("s") * _NC + lax.axis_index("c")
    pltpu.sync_copy(idx_hbm.at[pl.ds(wid * _NCH, _NCH)], idx_v)

    def group(g, carry):
        c0 = g * _K
        gd = []
        for b in range(_K):
            gd.append(pltpu.async_copy(
                table_hbm.at[idx_v.at[c0 + b]], rows_v.at[b], gsem))
        sd = []
        for b in range(_K):
            gd[b].wait()
            sd.append(pltpu.async_copy(
                rows_v.at[b],
                out_hbm.at[pl.ds((wid * _NCH + c0 + b) * _CHUNK, _CHUNK)],
                ssem))
        for b in range(_K):
            sd[b].wait()
        return carry

    lax.fori_loop(0, _NCH // _K, group, 0)


@functools.cache
def _sc_gather():
    mesh = plsc.VectorSubcoreMesh(
        core_axis_name="c", subcore_axis_name="s",
        num_cores=_NC, num_subcores=_NS)
    return pl.kernel(
        _sc_gather_body,
        out_type=jax.ShapeDtypeStruct((_ROWS, _EDP), jnp.float32),
        mesh=mesh,
        scratch_types=[
            pltpu.VMEM((_NCH, _CHUNK), jnp.int32),
            pltpu.VMEM((_K, _CHUNK, _EDP), jnp.float32),
            pltpu.SemaphoreType.DMA,
            pltpu.SemaphoreType.DMA,
        ],
        compiler_params=pltpu.CompilerParams(use_tc_tiling_on_sc=False),
    )


_BLK = 512
_E1 = _NF * _EDP
_ND = 13
_H1, _H2, _H3 = 512, 256, 128


def _mlp_body(emb_ref, num_ref, w1e_ref, w1n_ref, b1_ref, g1_ref, be1_ref,
              w2_ref, b2_ref, g2_ref, be2_ref,
              w3_ref, b3_ref, g3_ref, be3_ref,
              w4_ref, b4_ref, out_ref):
    h = jnp.dot(emb_ref[...], w1e_ref[...], preferred_element_type=jnp.float32)
    h = h + jnp.dot(num_ref[...], w1n_ref[...],
                    preferred_element_type=jnp.float32)
    h = ((h + b1_ref[...]) * _INV) * g1_ref[...] + be1_ref[...]
    x = jnp.maximum(h, 0.0)

    h = jnp.dot(x, w2_ref[...], preferred_element_type=jnp.float32)
    h = ((h + b2_ref[...]) * _INV) * g2_ref[...] + be2_ref[...]
    x = jnp.maximum(h, 0.0)

    h = jnp.dot(x, w3_ref[...], preferred_element_type=jnp.float32)
    h = ((h + b3_ref[...]) * _INV) * g3_ref[...] + be3_ref[...]
    x = jnp.maximum(h, 0.0)

    z = jnp.dot(x, w4_ref[...], preferred_element_type=jnp.float32)
    z = z + b4_ref[...]
    out_ref[...] = jax.nn.sigmoid(z)


def _full(shape):
    return pl.BlockSpec(shape, lambda i: (0, 0))


_mlp_call = pl.pallas_call(
    _mlp_body,
    grid=(_B // _BLK,),
    in_specs=[
        pl.BlockSpec((_BLK, _E1), lambda i: (i, 0)),
        pl.BlockSpec((_BLK, _ND), lambda i: (i, 0)),
        _full((_E1, _H1)), _full((_ND, _H1)),
        _full((1, _H1)), _full((1, _H1)), _full((1, _H1)),
        _full((_H1, _H2)), _full((1, _H2)), _full((1, _H2)), _full((1, _H2)),
        _full((_H2, _H3)), _full((1, _H3)), _full((1, _H3)), _full((1, _H3)),
        _full((_H3, 1)), _full((1, 1)),
    ],
    out_specs=pl.BlockSpec((_BLK, 1), lambda i: (i, 0)),
    out_shape=jax.ShapeDtypeStruct((_B, 1), jnp.float32),
)


def kernel(categorical_inputs, numerical_inputs, emb_tables,
           W1, b1, g1, be1, W2, b2, g2, be2, W3, b3, g3, be3, W4, b4):
    cat = jnp.clip(categorical_inputs, 0, _VOCAB - 1).astype(jnp.int32)
    idx = cat + (jnp.arange(_NF, dtype=jnp.int32) * _VOCAB)[None, :]
    idx2d = idx.reshape(_NW * _NCH, _CHUNK)

    table_lin = jnp.pad(
        emb_tables.reshape(_NF * _VOCAB, _ED),
        ((0, 0), (0, _EDP - _ED)))

    emb_flat = _sc_gather()(table_lin, idx2d)
    emb = emb_flat.reshape(_B, _E1)

    w1 = W1.T
    w1e = jnp.pad(w1[:_NF * _ED].reshape(_NF, _ED, _H1),
                  ((0, 0), (0, _EDP - _ED), (0, 0))).reshape(_E1, _H1)
    out = _mlp_call(
        emb, numerical_inputs,
        w1e, w1[_NF * _ED:],
        b1.reshape(1, _H1), g1.reshape(1, _H1), be1.reshape(1, _H1),
        W2.T, b2.reshape(1, _H2), g2.reshape(1, _H2), be2.reshape(1, _H2),
        W3.T, b3.reshape(1, _H3), g3.reshape(1, _H3), be3.reshape(1, _H3),
        W4.T, b4.reshape(1, 1),
    )
    return out[:, 0]
